# Initial kernel scaffold; baseline (speedup 1.0000x reference)
import jax
import jax.numpy as jnp
from jax.experimental import pallas as pl


def kernel(x, edge_index, edge_attr, W0, b0, A1, c1, A2, c2, Wroot, bconv, Wih, bih, Whh, bhh, W1, b1, W2, b2):
    raise NotImplementedError("write your pallas kernel here")


# speedup vs baseline: 1.4119x; 1.4119x over previous
import functools

import jax
import jax.numpy as jnp
from jax import lax
from jax.experimental import pallas as pl
from jax.experimental.pallas import tpu as pltpu
from jax.experimental.pallas import tpu_sc as plsc

N = 10000
E = 160000
IN_NF = 128
HID = 32
E_D = 4
MLP_H = 8
N_STEPS = 3
M9 = MLP_H + 1

NC = 2
NS = 16
NW = NC * NS
CH = 128
E_PAD = 163840
EPW = E_PAD // NW
NCHUNK = EPW // CH
N_ACC = 10240
RPT = N_ACC // NS

@functools.cache
def _sc_mesh():
    return plsc.VectorSubcoreMesh(core_axis_name="c", subcore_axis_name="s",
                                  num_cores=NC, num_subcores=NS)


def _kx0_body(x_ref, w0t_ref, b0_ref, o_ref):
    o_ref[...] = jnp.maximum(
        jnp.dot(x_ref[...], w0t_ref[...], preferred_element_type=jnp.float32, precision=lax.Precision.HIGHEST)
        + b0_ref[...], 0.0)


def _kx0(x, w0t, b0row):
    BN = 2000
    return pl.pallas_call(
        _kx0_body,
        grid=(N // BN,),
        in_specs=[
            pl.BlockSpec((BN, IN_NF), lambda i: (i, 0)),
            pl.BlockSpec((IN_NF, HID), lambda i: (0, 0)),
            pl.BlockSpec((1, HID), lambda i: (0, 0)),
        ],
        out_specs=pl.BlockSpec((BN, HID), lambda i: (i, 0)),
        out_shape=jax.ShapeDtypeStruct((N, HID), jnp.float32),
    )(x, w0t, b0row)


def _km_body(xg_ref, ea_ref, a1rept_ref, c1rep_ref, bcat_ref, summat_ref,
             o_ref):
    e1rep = jnp.maximum(
        jnp.dot(ea_ref[...], a1rept_ref[...],
                preferred_element_type=jnp.float32, precision=lax.Precision.HIGHEST) + c1rep_ref[...], 0.0)
    t = jnp.dot(xg_ref[...], bcat_ref[...],
                preferred_element_type=jnp.float32, precision=lax.Precision.HIGHEST)
    o_ref[...] = jnp.dot(e1rep * t, summat_ref[...],
                         preferred_element_type=jnp.float32, precision=lax.Precision.HIGHEST)


def _km(xg, eap, a1rept, c1rep, bcat, summat):
    BE = 2048
    return pl.pallas_call(
        _km_body,
        grid=(E_PAD // BE,),
        in_specs=[
            pl.BlockSpec((BE, HID), lambda i: (i, 0)),
            pl.BlockSpec((BE, E_D), lambda i: (i, 0)),
            pl.BlockSpec((E_D, M9 * HID), lambda i: (0, 0)),
            pl.BlockSpec((1, M9 * HID), lambda i: (0, 0)),
            pl.BlockSpec((HID, M9 * HID), lambda i: (0, 0)),
            pl.BlockSpec((M9 * HID, HID), lambda i: (0, 0)),
        ],
        out_specs=pl.BlockSpec((BE, HID), lambda i: (i, 0)),
        out_shape=jax.ShapeDtypeStruct((E_PAD, HID), jnp.float32),
    )(xg, eap, a1rept, c1rep, bcat, summat)


def _kn_body(p_ref, d_ref, xx_ref, wroott_ref, bconv_ref,
             wiht_ref, bih_ref, whht_ref, bhh_ref, o_ref):
    xx = xx_ref[...]
    deg = jnp.maximum(d_ref[0] + d_ref[1], 1.0)
    agg = (p_ref[0] + p_ref[1]) / deg
    m = jnp.maximum(
        agg + jnp.dot(xx, wroott_ref[...], preferred_element_type=jnp.float32, precision=lax.Precision.HIGHEST)
        + bconv_ref[...], 0.0)
    gi = jnp.dot(m, wiht_ref[...], preferred_element_type=jnp.float32, precision=lax.Precision.HIGHEST) \
        + bih_ref[...]
    gh = jnp.dot(xx, whht_ref[...], preferred_element_type=jnp.float32, precision=lax.Precision.HIGHEST) \
        + bhh_ref[...]
    r = jax.nn.sigmoid(gi[:, :HID] + gh[:, :HID])
    z = jax.nn.sigmoid(gi[:, HID:2 * HID] + gh[:, HID:2 * HID])
    nn = jnp.tanh(gi[:, 2 * HID:] + r * gh[:, 2 * HID:])
    o_ref[...] = (1.0 - z) * nn + z * xx


def _kn(p, d, xx, wroott, bconvrow, wiht, bihrow, whht, bhhrow):
    BN = 2000
    return pl.pallas_call(
        _kn_body,
        grid=(N // BN,),
        in_specs=[
            pl.BlockSpec((2, BN, HID), lambda i: (0, i, 0)),
            pl.BlockSpec((2, BN, HID), lambda i: (0, i, 0)),
            pl.BlockSpec((BN, HID), lambda i: (i, 0)),
            pl.BlockSpec((HID, HID), lambda i: (0, 0)),
            pl.BlockSpec((1, HID), lambda i: (0, 0)),
            pl.BlockSpec((HID, 3 * HID), lambda i: (0, 0)),
            pl.BlockSpec((1, 3 * HID), lambda i: (0, 0)),
            pl.BlockSpec((HID, 3 * HID), lambda i: (0, 0)),
            pl.BlockSpec((1, 3 * HID), lambda i: (0, 0)),
        ],
        out_specs=pl.BlockSpec((BN, HID), lambda i: (i, 0)),
        out_shape=jax.ShapeDtypeStruct((N, HID), jnp.float32),
    )(p, d, xx, wroott, bconvrow, wiht, bihrow, whht, bhhrow)


def _khead_body(xx_ref, w1t_ref, b1_ref, w2t_ref, b2_ref, o_ref):
    g = jnp.mean(xx_ref[...], axis=0, keepdims=True)
    g = jnp.maximum(
        jnp.dot(g, w1t_ref[...], preferred_element_type=jnp.float32, precision=lax.Precision.HIGHEST)
        + b1_ref[...], 0.0)
    o_ref[...] = jnp.dot(g, w2t_ref[...],
                         preferred_element_type=jnp.float32, precision=lax.Precision.HIGHEST) + b2_ref[...]


def _khead(xx, w1t, b1row, w2t, b2row):
    return pl.pallas_call(
        _khead_body,
        out_shape=jax.ShapeDtypeStruct((1, 1), jnp.float32),
    )(xx, w1t, b1row, w2t, b2row)


def _worker_id():
    return lax.axis_index("s") * NC + lax.axis_index("c")


def _kg_body(xx_hbm, src3_hbm, out_hbm, idx_v, rows_v, gsem, ssem):
    wid = _worker_id()
    pltpu.sync_copy(src3_hbm.at[wid], idx_v)

    def chunk(j, carry):
        pltpu.async_copy(xx_hbm.at[idx_v.at[j]], rows_v, gsem).wait()
        pltpu.async_copy(
            rows_v, out_hbm.at[pl.ds(wid * EPW + j * CH, CH)], ssem).wait()
        return carry

    lax.fori_loop(0, NCHUNK, chunk, 0)


@functools.cache
def _kg_fn():
    return pl.kernel(
        _kg_body,
        out_type=jax.ShapeDtypeStruct((E_PAD, HID), jnp.float32),
        mesh=_sc_mesh(),
        compiler_params=pltpu.CompilerParams(use_tc_tiling_on_sc=False),
        scratch_types=[
            pltpu.VMEM((NCHUNK, CH), jnp.int32),
            pltpu.VMEM((CH, HID), jnp.float32),
            pltpu.SemaphoreType.DMA,
            pltpu.SemaphoreType.DMA,
        ],
    )


def _kg(xx, src3):
    return _kg_fn()(xx, src3)


def _ks_body(msg_hbm, dst3_hbm, zeros_hbm, out_hbm, idx_v, rows_v, acc_sh,
             lsem):
    cid = lax.axis_index("c")
    sid = lax.axis_index("s")
    wid = sid * NC + cid
    pltpu.sync_copy(zeros_hbm.at[pl.ds(sid * RPT, RPT)],
                    acc_sh.at[pl.ds(sid * RPT, RPT)])
    pltpu.sync_copy(dst3_hbm.at[wid], idx_v)
    plsc.subcore_barrier()

    def chunk(j, carry):
        pltpu.async_copy(
            msg_hbm.at[pl.ds(wid * EPW + j * CH, CH)], rows_v, lsem).wait()
        pltpu.sync_copy(rows_v, acc_sh.at[idx_v.at[j]], add=True)
        return carry

    lax.fori_loop(0, NCHUNK, chunk, 0)
    plsc.subcore_barrier()
    pltpu.sync_copy(acc_sh.at[pl.ds(sid * RPT, RPT)],
                    out_hbm.at[cid, pl.ds(sid * RPT, RPT)])


@functools.cache
def _ks_fn():
    return pl.kernel(
        _ks_body,
        out_type=jax.ShapeDtypeStruct((NC, N_ACC, HID), jnp.float32),
        mesh=_sc_mesh(),
        compiler_params=pltpu.CompilerParams(use_tc_tiling_on_sc=False),
        scratch_types=[
            pltpu.VMEM((NCHUNK, CH), jnp.int32),
            pltpu.VMEM((CH, HID), jnp.float32),
            pltpu.VMEM_SHARED((N_ACC, HID), jnp.float32),
            pltpu.SemaphoreType.DMA,
        ],
    )


def _ks(msg, dst3, zeros_acc):
    return _ks_fn()(msg, dst3, zeros_acc)


def _kdeg_body(ones_hbm, dst3_hbm, zeros_hbm, out_hbm, idx_v, rows_v, acc_sh):
    cid = lax.axis_index("c")
    sid = lax.axis_index("s")
    wid = sid * NC + cid
    pltpu.sync_copy(zeros_hbm.at[pl.ds(sid * RPT, RPT)],
                    acc_sh.at[pl.ds(sid * RPT, RPT)])
    pltpu.sync_copy(dst3_hbm.at[wid], idx_v)
    pltpu.sync_copy(ones_hbm, rows_v)
    plsc.subcore_barrier()

    def chunk(j, carry):
        pltpu.sync_copy(rows_v, acc_sh.at[idx_v.at[j]], add=True)
        return carry

    lax.fori_loop(0, NCHUNK, chunk, 0)
    plsc.subcore_barrier()
    pltpu.sync_copy(acc_sh.at[pl.ds(sid * RPT, RPT)],
                    out_hbm.at[cid, pl.ds(sid * RPT, RPT)])


@functools.cache
def _kdeg_fn():
    return pl.kernel(
        _kdeg_body,
        out_type=jax.ShapeDtypeStruct((NC, N_ACC, HID), jnp.float32),
        mesh=_sc_mesh(),
        compiler_params=pltpu.CompilerParams(use_tc_tiling_on_sc=False),
        scratch_types=[
            pltpu.VMEM((NCHUNK, CH), jnp.int32),
            pltpu.VMEM((CH, HID), jnp.float32),
            pltpu.VMEM_SHARED((N_ACC, HID), jnp.float32),
        ],
    )


def _kdeg(ones_rows, dst3, zeros_acc):
    return _kdeg_fn()(ones_rows, dst3, zeros_acc)


def kernel(x, edge_index, edge_attr, W0, b0, A1, c1, A2, c2, Wroot, bconv,
           Wih, bih, Whh, bhh, W1, b1, W2, b2):
    f32 = jnp.float32
    src = edge_index[0]
    dst = edge_index[1]
    pad = E_PAD - E
    srcp = jnp.pad(src, (0, pad))
    dstp = jnp.pad(dst, (0, pad), constant_values=N)
    eap = jnp.pad(edge_attr, ((0, pad), (0, 0)))
    src3 = srcp.reshape(NW, NCHUNK, CH)
    dst3 = dstp.reshape(NW, NCHUNK, CH)

    w0t = W0.T
    b0row = b0.reshape(1, HID)
    rt = A2.reshape(N_STEPS, HID, HID, MLP_H)
    ball = jnp.concatenate(
        [rt, c2.reshape(N_STEPS, HID, HID)[..., None]], axis=-1)
    bcat = ball.transpose(0, 1, 3, 2).reshape(N_STEPS, HID, M9 * HID)
    a1aug = jnp.concatenate(
        [jnp.swapaxes(A1, 1, 2), jnp.zeros((N_STEPS, E_D, 1), f32)], axis=2)
    c1aug = jnp.concatenate(
        [c1, jnp.ones((N_STEPS, 1), f32)], axis=1)
    a1rept = jnp.repeat(a1aug, HID, axis=2)
    c1rep = jnp.repeat(c1aug, HID, axis=1)[:, None, :]
    summat = jnp.tile(jnp.eye(HID, dtype=f32), (M9, 1))
    wroott = jnp.swapaxes(Wroot, 1, 2)
    bconvrow = bconv.reshape(N_STEPS, 1, HID)
    wiht = Wih.T
    bihrow = bih.reshape(1, 3 * HID)
    whht = Whh.T
    bhhrow = bhh.reshape(1, 3 * HID)
    w1t = W1.T
    b1row = b1.reshape(1, HID // 2)
    w2t = W2.T
    b2row = b2.reshape(1, 1)

    zeros_acc = jnp.zeros((N_ACC, HID), f32)
    ones_rows = jnp.ones((CH, HID), f32)

    xx = _kx0(x, w0t, b0row)
    d = _kdeg(ones_rows, dst3, zeros_acc)
    for l in range(N_STEPS):
        xg = _kg(xx, src3)
        msg = _km(xg, eap, a1rept[l], c1rep[l], bcat[l], summat)
        p = _ks(msg, dst3, zeros_acc)
        xx = _kn(p, d, xx, wroott[l], bconvrow[l], wiht, bihrow, whht,
                 bhhrow)
    return _khead(xx, w1t, b1row, w2t, b2row)


# speedup vs baseline: 1.4742x; 1.0441x over previous
import functools

import jax
import jax.numpy as jnp
from jax import lax
from jax.experimental import pallas as pl
from jax.experimental.pallas import tpu as pltpu
from jax.experimental.pallas import tpu_sc as plsc

N = 10000
E = 160000
IN_NF = 128
HID = 32
E_D = 4
MLP_H = 8
N_STEPS = 3
M9 = MLP_H + 1

NC = 2
NS = 16
NW = NC * NS
CH = 128
E_PAD = 163840
EPW = E_PAD // NW
NCHUNK = EPW // CH
N_ACC = 10240
RPT = N_ACC // NS

@functools.cache
def _sc_mesh():
    return plsc.VectorSubcoreMesh(core_axis_name="c", subcore_axis_name="s",
                                  num_cores=NC, num_subcores=NS)


def _kx0_body(x_ref, w0t_ref, b0_ref, o_ref):
    o_ref[...] = jnp.maximum(
        jnp.dot(x_ref[...], w0t_ref[...], preferred_element_type=jnp.float32, precision=lax.Precision.HIGHEST)
        + b0_ref[...], 0.0)


def _kx0(x, w0t, b0row):
    BN = 2000
    return pl.pallas_call(
        _kx0_body,
        grid=(N // BN,),
        in_specs=[
            pl.BlockSpec((BN, IN_NF), lambda i: (i, 0)),
            pl.BlockSpec((IN_NF, HID), lambda i: (0, 0)),
            pl.BlockSpec((1, HID), lambda i: (0, 0)),
        ],
        out_specs=pl.BlockSpec((BN, HID), lambda i: (i, 0)),
        out_shape=jax.ShapeDtypeStruct((N, HID), jnp.float32),
    )(x, w0t, b0row)


def _km_body(xg_ref, ea_ref, a1rept_ref, c1rep_ref, bcat_ref, summat_ref,
             o_ref):
    e1rep = jnp.maximum(
        jnp.dot(ea_ref[...], a1rept_ref[...],
                preferred_element_type=jnp.float32, precision=lax.Precision.HIGHEST) + c1rep_ref[...], 0.0)
    t = jnp.dot(xg_ref[...], bcat_ref[...],
                preferred_element_type=jnp.float32, precision=lax.Precision.HIGHEST)
    o_ref[...] = jnp.dot(e1rep * t, summat_ref[...],
                         preferred_element_type=jnp.float32, precision=lax.Precision.HIGHEST)


def _km(xg, eap, a1rept, c1rep, bcat, summat):
    BE = 2048
    return pl.pallas_call(
        _km_body,
        grid=(E_PAD // BE,),
        in_specs=[
            pl.BlockSpec((BE, HID), lambda i: (i, 0)),
            pl.BlockSpec((BE, E_D), lambda i: (i, 0)),
            pl.BlockSpec((E_D, M9 * HID), lambda i: (0, 0)),
            pl.BlockSpec((1, M9 * HID), lambda i: (0, 0)),
            pl.BlockSpec((HID, M9 * HID), lambda i: (0, 0)),
            pl.BlockSpec((M9 * HID, HID), lambda i: (0, 0)),
        ],
        out_specs=pl.BlockSpec((BE, HID), lambda i: (i, 0)),
        out_shape=jax.ShapeDtypeStruct((E_PAD, HID), jnp.float32),
    )(xg, eap, a1rept, c1rep, bcat, summat)


def _kn_body(p_ref, d_ref, xx_ref, wroott_ref, bconv_ref,
             wiht_ref, bih_ref, whht_ref, bhh_ref, o_ref):
    xx = xx_ref[...]
    deg = jnp.maximum(d_ref[0] + d_ref[1], 1.0)
    agg = (p_ref[0] + p_ref[1]) / deg
    m = jnp.maximum(
        agg + jnp.dot(xx, wroott_ref[...], preferred_element_type=jnp.float32, precision=lax.Precision.HIGHEST)
        + bconv_ref[...], 0.0)
    gi = jnp.dot(m, wiht_ref[...], preferred_element_type=jnp.float32, precision=lax.Precision.HIGHEST) \
        + bih_ref[...]
    gh = jnp.dot(xx, whht_ref[...], preferred_element_type=jnp.float32, precision=lax.Precision.HIGHEST) \
        + bhh_ref[...]
    r = jax.nn.sigmoid(gi[:, :HID] + gh[:, :HID])
    z = jax.nn.sigmoid(gi[:, HID:2 * HID] + gh[:, HID:2 * HID])
    nn = jnp.tanh(gi[:, 2 * HID:] + r * gh[:, 2 * HID:])
    o_ref[...] = (1.0 - z) * nn + z * xx


def _kn(p, d, xx, wroott, bconvrow, wiht, bihrow, whht, bhhrow):
    BN = 2000
    return pl.pallas_call(
        _kn_body,
        grid=(N // BN,),
        in_specs=[
            pl.BlockSpec((2, BN, HID), lambda i: (0, i, 0)),
            pl.BlockSpec((2, BN, HID), lambda i: (0, i, 0)),
            pl.BlockSpec((BN, HID), lambda i: (i, 0)),
            pl.BlockSpec((HID, HID), lambda i: (0, 0)),
            pl.BlockSpec((1, HID), lambda i: (0, 0)),
            pl.BlockSpec((HID, 3 * HID), lambda i: (0, 0)),
            pl.BlockSpec((1, 3 * HID), lambda i: (0, 0)),
            pl.BlockSpec((HID, 3 * HID), lambda i: (0, 0)),
            pl.BlockSpec((1, 3 * HID), lambda i: (0, 0)),
        ],
        out_specs=pl.BlockSpec((BN, HID), lambda i: (i, 0)),
        out_shape=jax.ShapeDtypeStruct((N, HID), jnp.float32),
    )(p, d, xx, wroott, bconvrow, wiht, bihrow, whht, bhhrow)


def _khead_body(xx_ref, w1t_ref, b1_ref, w2t_ref, b2_ref, o_ref):
    g = jnp.mean(xx_ref[...], axis=0, keepdims=True)
    g = jnp.maximum(
        jnp.dot(g, w1t_ref[...], preferred_element_type=jnp.float32, precision=lax.Precision.HIGHEST)
        + b1_ref[...], 0.0)
    o_ref[...] = jnp.dot(g, w2t_ref[...],
                         preferred_element_type=jnp.float32, precision=lax.Precision.HIGHEST) + b2_ref[...]


def _khead(xx, w1t, b1row, w2t, b2row):
    return pl.pallas_call(
        _khead_body,
        out_shape=jax.ShapeDtypeStruct((1, 1), jnp.float32),
    )(xx, w1t, b1row, w2t, b2row)


def _worker_id():
    return lax.axis_index("s") * NC + lax.axis_index("c")


NBUF = 4
NGRP = NCHUNK // NBUF


def _kg_body(xx_hbm, src3_hbm, out_hbm, idx_v,
             r0, r1, r2, r3, g0, g1, g2, g3, s0, s1, s2, s3):
    rows = (r0, r1, r2, r3)
    gs = (g0, g1, g2, g3)
    ss = (s0, s1, s2, s3)
    wid = _worker_id()
    base = wid * EPW
    pltpu.sync_copy(src3_hbm.at[wid], idx_v)

    def g_start(j, b):
        pltpu.make_async_copy(xx_hbm.at[idx_v.at[j]], rows[b], gs[b]).start()

    def g_wait(b):
        pltpu.make_async_copy(xx_hbm.at[idx_v.at[0]], rows[b], gs[b]).wait()

    def s_start(j, b):
        pltpu.make_async_copy(
            rows[b], out_hbm.at[pl.ds(base + j * CH, CH)], ss[b]).start()

    def s_wait(b):
        pltpu.make_async_copy(
            rows[b], out_hbm.at[pl.ds(base, CH)], ss[b]).wait()

    for b in range(NBUF):
        g_start(b, b)

    def group(g, carry):
        for b in range(NBUF):
            g_wait(b)
            s_start(g * NBUF + b, b)

        @pl.when(g < NGRP - 1)
        def _():
            for b in range(NBUF):
                s_wait(b)
                g_start((g + 1) * NBUF + b, b)

        return carry

    lax.fori_loop(0, NGRP, group, 0)
    for b in range(NBUF):
        s_wait(b)


@functools.cache
def _kg_fn():
    return pl.kernel(
        _kg_body,
        out_type=jax.ShapeDtypeStruct((E_PAD, HID), jnp.float32),
        mesh=_sc_mesh(),
        compiler_params=pltpu.CompilerParams(use_tc_tiling_on_sc=False),
        scratch_types=[pltpu.VMEM((NCHUNK, CH), jnp.int32)]
        + [pltpu.VMEM((CH, HID), jnp.float32)] * NBUF
        + [pltpu.SemaphoreType.DMA] * (2 * NBUF),
    )


def _kg(xx, src3):
    return _kg_fn()(xx, src3)


def _ks_body(msg_hbm, dst3_hbm, zeros_hbm, out_hbm, idx_v,
             r0, r1, r2, r3, l0, l1, l2, l3, a0, a1, a2, a3, acc_sh):
    rows = (r0, r1, r2, r3)
    ls = (l0, l1, l2, l3)
    as_ = (a0, a1, a2, a3)
    cid = lax.axis_index("c")
    sid = lax.axis_index("s")
    wid = sid * NC + cid
    base = wid * EPW
    pltpu.sync_copy(zeros_hbm.at[pl.ds(sid * RPT, RPT)],
                    acc_sh.at[pl.ds(sid * RPT, RPT)])
    pltpu.sync_copy(dst3_hbm.at[wid], idx_v)
    plsc.subcore_barrier()

    def l_start(j, b):
        pltpu.make_async_copy(
            msg_hbm.at[pl.ds(base + j * CH, CH)], rows[b], ls[b]).start()

    def l_wait(b):
        pltpu.make_async_copy(
            msg_hbm.at[pl.ds(base, CH)], rows[b], ls[b]).wait()

    def a_start(j, b):
        pltpu.make_async_copy(
            rows[b], acc_sh.at[idx_v.at[j]], as_[b]).start(add=True)

    def a_wait(b):
        pltpu.make_async_copy(
            rows[b], acc_sh.at[idx_v.at[0]], as_[b]).wait()

    for b in range(NBUF):
        l_start(b, b)

    def group(g, carry):
        for b in range(NBUF):
            l_wait(b)
            a_start(g * NBUF + b, b)

        @pl.when(g < NGRP - 1)
        def _():
            for b in range(NBUF):
                a_wait(b)
                l_start((g + 1) * NBUF + b, b)

        return carry

    lax.fori_loop(0, NGRP, group, 0)
    for b in range(NBUF):
        a_wait(b)
    plsc.subcore_barrier()
    pltpu.sync_copy(acc_sh.at[pl.ds(sid * RPT, RPT)],
                    out_hbm.at[cid, pl.ds(sid * RPT, RPT)])


@functools.cache
def _ks_fn():
    return pl.kernel(
        _ks_body,
        out_type=jax.ShapeDtypeStruct((NC, N_ACC, HID), jnp.float32),
        mesh=_sc_mesh(),
        compiler_params=pltpu.CompilerParams(use_tc_tiling_on_sc=False),
        scratch_types=[pltpu.VMEM((NCHUNK, CH), jnp.int32)]
        + [pltpu.VMEM((CH, HID), jnp.float32)] * NBUF
        + [pltpu.SemaphoreType.DMA] * (2 * NBUF)
        + [pltpu.VMEM_SHARED((N_ACC, HID), jnp.float32)],
    )


def _ks(msg, dst3, zeros_acc):
    return _ks_fn()(msg, dst3, zeros_acc)


def _kdeg_body(ones_hbm, dst3_hbm, zeros_hbm, out_hbm, idx_v, rows_v, asem,
               acc_sh):
    cid = lax.axis_index("c")
    sid = lax.axis_index("s")
    wid = sid * NC + cid
    pltpu.sync_copy(zeros_hbm.at[pl.ds(sid * RPT, RPT)],
                    acc_sh.at[pl.ds(sid * RPT, RPT)])
    pltpu.sync_copy(dst3_hbm.at[wid], idx_v)
    pltpu.sync_copy(ones_hbm, rows_v)
    plsc.subcore_barrier()

    def fire(j, carry):
        pltpu.make_async_copy(
            rows_v, acc_sh.at[idx_v.at[j]], asem).start(add=True)
        return carry

    def drain(j, carry):
        pltpu.make_async_copy(rows_v, acc_sh.at[idx_v.at[0]], asem).wait()
        return carry

    lax.fori_loop(0, NCHUNK, fire, 0)
    lax.fori_loop(0, NCHUNK, drain, 0)
    plsc.subcore_barrier()
    pltpu.sync_copy(acc_sh.at[pl.ds(sid * RPT, RPT)],
                    out_hbm.at[cid, pl.ds(sid * RPT, RPT)])


@functools.cache
def _kdeg_fn():
    return pl.kernel(
        _kdeg_body,
        out_type=jax.ShapeDtypeStruct((NC, N_ACC, HID), jnp.float32),
        mesh=_sc_mesh(),
        compiler_params=pltpu.CompilerParams(use_tc_tiling_on_sc=False),
        scratch_types=[
            pltpu.VMEM((NCHUNK, CH), jnp.int32),
            pltpu.VMEM((CH, HID), jnp.float32),
            pltpu.SemaphoreType.DMA,
            pltpu.VMEM_SHARED((N_ACC, HID), jnp.float32),
        ],
    )


def _kdeg(ones_rows, dst3, zeros_acc):
    return _kdeg_fn()(ones_rows, dst3, zeros_acc)


def kernel(x, edge_index, edge_attr, W0, b0, A1, c1, A2, c2, Wroot, bconv,
           Wih, bih, Whh, bhh, W1, b1, W2, b2):
    f32 = jnp.float32
    src = edge_index[0]
    dst = edge_index[1]
    pad = E_PAD - E
    srcp = jnp.pad(src, (0, pad))
    dstp = jnp.pad(dst, (0, pad), constant_values=N)
    eap = jnp.pad(edge_attr, ((0, pad), (0, 0)))
    src3 = srcp.reshape(NW, NCHUNK, CH)
    dst3 = dstp.reshape(NW, NCHUNK, CH)

    w0t = W0.T
    b0row = b0.reshape(1, HID)
    rt = A2.reshape(N_STEPS, HID, HID, MLP_H)
    ball = jnp.concatenate(
        [rt, c2.reshape(N_STEPS, HID, HID)[..., None]], axis=-1)
    bcat = ball.transpose(0, 1, 3, 2).reshape(N_STEPS, HID, M9 * HID)
    a1aug = jnp.concatenate(
        [jnp.swapaxes(A1, 1, 2), jnp.zeros((N_STEPS, E_D, 1), f32)], axis=2)
    c1aug = jnp.concatenate(
        [c1, jnp.ones((N_STEPS, 1), f32)], axis=1)
    a1rept = jnp.repeat(a1aug, HID, axis=2)
    c1rep = jnp.repeat(c1aug, HID, axis=1)[:, None, :]
    summat = jnp.tile(jnp.eye(HID, dtype=f32), (M9, 1))
    wroott = jnp.swapaxes(Wroot, 1, 2)
    bconvrow = bconv.reshape(N_STEPS, 1, HID)
    wiht = Wih.T
    bihrow = bih.reshape(1, 3 * HID)
    whht = Whh.T
    bhhrow = bhh.reshape(1, 3 * HID)
    w1t = W1.T
    b1row = b1.reshape(1, HID // 2)
    w2t = W2.T
    b2row = b2.reshape(1, 1)

    zeros_acc = jnp.zeros((N_ACC, HID), f32)
    ones_rows = jnp.ones((CH, HID), f32)

    xx = _kx0(x, w0t, b0row)
    d = _kdeg(ones_rows, dst3, zeros_acc)
    for l in range(N_STEPS):
        xg = _kg(xx, src3)
        msg = _km(xg, eap, a1rept[l], c1rep[l], bcat[l], summat)
        p = _ks(msg, dst3, zeros_acc)
        xx = _kn(p, d, xx, wroott[l], bconvrow[l], wiht, bihrow, whht,
                 bhhrow)
    return _khead(xx, w1t, b1row, w2t, b2row)


# speedup vs baseline: 1.8735x; 1.2708x over previous
import functools

import jax
import jax.numpy as jnp
from jax import lax
from jax.experimental import pallas as pl
from jax.experimental.pallas import tpu as pltpu
from jax.experimental.pallas import tpu_sc as plsc

N = 10000
E = 160000
IN_NF = 128
HID = 32
E_D = 4
MLP_H = 8
N_STEPS = 3
M9 = MLP_H + 1

NC = 2
NS = 16
NW = NC * NS
CH = 128
E_PAD = 163840
EPW = E_PAD // NW
NCHUNK = EPW // CH
N_ACC = 10240
RPT = N_ACC // NS

@functools.cache
def _sc_mesh():
    return plsc.VectorSubcoreMesh(core_axis_name="c", subcore_axis_name="s",
                                  num_cores=NC, num_subcores=NS)


def _kx0_body(x_ref, w0t_ref, b0_ref, o_ref):
    o_ref[...] = jnp.maximum(
        jnp.dot(x_ref[...], w0t_ref[...], preferred_element_type=jnp.float32, precision=lax.Precision.HIGHEST)
        + b0_ref[...], 0.0)


def _kx0(x, w0t, b0row):
    BN = 2000
    return pl.pallas_call(
        _kx0_body,
        grid=(N // BN,),
        in_specs=[
            pl.BlockSpec((BN, IN_NF), lambda i: (i, 0)),
            pl.BlockSpec((IN_NF, HID), lambda i: (0, 0)),
            pl.BlockSpec((1, HID), lambda i: (0, 0)),
        ],
        out_specs=pl.BlockSpec((BN, HID), lambda i: (i, 0)),
        out_shape=jax.ShapeDtypeStruct((N, HID), jnp.float32),
    )(x, w0t, b0row)


def _km_body(xg_ref, ea_ref, a1rept_ref, c1rep_ref, bcat_ref, o_ref):
    e1rep = jnp.maximum(
        jnp.dot(ea_ref[...], a1rept_ref[...],
                preferred_element_type=jnp.float32, precision=lax.Precision.HIGHEST) + c1rep_ref[...], 0.0)
    t = e1rep * jnp.dot(xg_ref[...], bcat_ref[...],
                        preferred_element_type=jnp.float32, precision=lax.Precision.HIGHEST)
    acc = t[:, :HID]
    for m in range(1, M9):
        acc = acc + t[:, m * HID:(m + 1) * HID]
    o_ref[...] = acc


def _km(xg, eap, a1rept, c1rep, bcat):
    BE = 2048
    return pl.pallas_call(
        _km_body,
        grid=(E_PAD // BE,),
        in_specs=[
            pl.BlockSpec((BE, HID), lambda i: (i, 0)),
            pl.BlockSpec((BE, E_D), lambda i: (i, 0)),
            pl.BlockSpec((E_D, M9 * HID), lambda i: (0, 0)),
            pl.BlockSpec((1, M9 * HID), lambda i: (0, 0)),
            pl.BlockSpec((HID, M9 * HID), lambda i: (0, 0)),
        ],
        out_specs=pl.BlockSpec((BE, HID), lambda i: (i, 0)),
        out_shape=jax.ShapeDtypeStruct((E_PAD, HID), jnp.float32),
    )(xg, eap, a1rept, c1rep, bcat)


def _kn_body(p_ref, d_ref, xx_ref, wroott_ref, bconv_ref,
             wiht_ref, bih_ref, whht_ref, bhh_ref, o_ref):
    xx = xx_ref[...]
    deg = jnp.maximum(d_ref[0] + d_ref[1], 1.0)
    agg = (p_ref[0] + p_ref[1]) / deg
    m = jnp.maximum(
        agg + jnp.dot(xx, wroott_ref[...], preferred_element_type=jnp.float32, precision=lax.Precision.HIGHEST)
        + bconv_ref[...], 0.0)
    gi = jnp.dot(m, wiht_ref[...], preferred_element_type=jnp.float32, precision=lax.Precision.HIGHEST) \
        + bih_ref[...]
    gh = jnp.dot(xx, whht_ref[...], preferred_element_type=jnp.float32, precision=lax.Precision.HIGHEST) \
        + bhh_ref[...]
    r = jax.nn.sigmoid(gi[:, :HID] + gh[:, :HID])
    z = jax.nn.sigmoid(gi[:, HID:2 * HID] + gh[:, HID:2 * HID])
    nn = jnp.tanh(gi[:, 2 * HID:] + r * gh[:, 2 * HID:])
    o_ref[...] = (1.0 - z) * nn + z * xx


def _kn(p, d, xx, wroott, bconvrow, wiht, bihrow, whht, bhhrow):
    BN = 2000
    return pl.pallas_call(
        _kn_body,
        grid=(N // BN,),
        in_specs=[
            pl.BlockSpec((2, BN, HID), lambda i: (0, i, 0)),
            pl.BlockSpec((2, BN, HID), lambda i: (0, i, 0)),
            pl.BlockSpec((BN, HID), lambda i: (i, 0)),
            pl.BlockSpec((HID, HID), lambda i: (0, 0)),
            pl.BlockSpec((1, HID), lambda i: (0, 0)),
            pl.BlockSpec((HID, 3 * HID), lambda i: (0, 0)),
            pl.BlockSpec((1, 3 * HID), lambda i: (0, 0)),
            pl.BlockSpec((HID, 3 * HID), lambda i: (0, 0)),
            pl.BlockSpec((1, 3 * HID), lambda i: (0, 0)),
        ],
        out_specs=pl.BlockSpec((BN, HID), lambda i: (i, 0)),
        out_shape=jax.ShapeDtypeStruct((N, HID), jnp.float32),
    )(p, d, xx, wroott, bconvrow, wiht, bihrow, whht, bhhrow)


def _khead_body(xx_ref, w1t_ref, b1_ref, w2t_ref, b2_ref, o_ref):
    g = jnp.mean(xx_ref[...], axis=0, keepdims=True)
    g = jnp.maximum(
        jnp.dot(g, w1t_ref[...], preferred_element_type=jnp.float32, precision=lax.Precision.HIGHEST)
        + b1_ref[...], 0.0)
    o_ref[...] = jnp.dot(g, w2t_ref[...],
                         preferred_element_type=jnp.float32, precision=lax.Precision.HIGHEST) + b2_ref[...]


def _khead(xx, w1t, b1row, w2t, b2row):
    return pl.pallas_call(
        _khead_body,
        out_shape=jax.ShapeDtypeStruct((1, 1), jnp.float32),
    )(xx, w1t, b1row, w2t, b2row)


def _worker_id():
    return lax.axis_index("s") * NC + lax.axis_index("c")


NBUF = 4
NGRP = NCHUNK // NBUF


def _kg_body(xx_hbm, src3_hbm, out_hbm, idx_v,
             r0, r1, r2, r3, g0, g1, g2, g3, s0, s1, s2, s3):
    rows = (r0, r1, r2, r3)
    gs = (g0, g1, g2, g3)
    ss = (s0, s1, s2, s3)
    wid = _worker_id()
    base = wid * EPW
    pltpu.sync_copy(src3_hbm.at[wid], idx_v)

    def g_start(j, b):
        pltpu.make_async_copy(xx_hbm.at[idx_v.at[j]], rows[b], gs[b]).start()

    def g_wait(b):
        pltpu.make_async_copy(xx_hbm.at[idx_v.at[0]], rows[b], gs[b]).wait()

    def s_start(j, b):
        pltpu.make_async_copy(
            rows[b], out_hbm.at[pl.ds(base + j * CH, CH)], ss[b]).start()

    def s_wait(b):
        pltpu.make_async_copy(
            rows[b], out_hbm.at[pl.ds(base, CH)], ss[b]).wait()

    for b in range(NBUF):
        g_start(b, b)

    def group(g, carry):
        for b in range(NBUF):
            g_wait(b)
            s_start(g * NBUF + b, b)

        @pl.when(g < NGRP - 1)
        def _():
            for b in range(NBUF):
                s_wait(b)
                g_start((g + 1) * NBUF + b, b)

        return carry

    lax.fori_loop(0, NGRP, group, 0)
    for b in range(NBUF):
        s_wait(b)


@functools.cache
def _kg_fn():
    return pl.kernel(
        _kg_body,
        out_type=jax.ShapeDtypeStruct((E_PAD, HID), jnp.float32),
        mesh=_sc_mesh(),
        compiler_params=pltpu.CompilerParams(use_tc_tiling_on_sc=False),
        scratch_types=[pltpu.VMEM((NCHUNK, CH), jnp.int32)]
        + [pltpu.VMEM((CH, HID), jnp.float32)] * NBUF
        + [pltpu.SemaphoreType.DMA] * (2 * NBUF),
    )


def _kg(xx, src3):
    return _kg_fn()(xx, src3)


def _ks_body(msg_hbm, dst3_hbm, zeros_hbm, out_hbm, idx_v,
             r0, r1, r2, r3, l0, l1, l2, l3, a0, a1, a2, a3, acc_sh):
    rows = (r0, r1, r2, r3)
    ls = (l0, l1, l2, l3)
    as_ = (a0, a1, a2, a3)
    cid = lax.axis_index("c")
    sid = lax.axis_index("s")
    wid = sid * NC + cid
    base = wid * EPW
    pltpu.sync_copy(zeros_hbm.at[pl.ds(sid * RPT, RPT)],
                    acc_sh.at[pl.ds(sid * RPT, RPT)])
    pltpu.sync_copy(dst3_hbm.at[wid], idx_v)
    plsc.subcore_barrier()

    def l_start(j, b):
        pltpu.make_async_copy(
            msg_hbm.at[pl.ds(base + j * CH, CH)], rows[b], ls[b]).start()

    def l_wait(b):
        pltpu.make_async_copy(
            msg_hbm.at[pl.ds(base, CH)], rows[b], ls[b]).wait()

    def a_start(j, b):
        pltpu.make_async_copy(
            rows[b], acc_sh.at[idx_v.at[j]], as_[b]).start(add=True)

    def a_wait(b):
        pltpu.make_async_copy(
            rows[b], acc_sh.at[idx_v.at[0]], as_[b]).wait()

    for b in range(NBUF):
        l_start(b, b)

    def group(g, carry):
        for b in range(NBUF):
            l_wait(b)
            a_start(g * NBUF + b, b)

        @pl.when(g < NGRP - 1)
        def _():
            for b in range(NBUF):
                a_wait(b)
                l_start((g + 1) * NBUF + b, b)

        return carry

    lax.fori_loop(0, NGRP, group, 0)
    for b in range(NBUF):
        a_wait(b)
    plsc.subcore_barrier()
    pltpu.sync_copy(acc_sh.at[pl.ds(sid * RPT, RPT)],
                    out_hbm.at[cid, pl.ds(sid * RPT, RPT)])


@functools.cache
def _ks_fn():
    return pl.kernel(
        _ks_body,
        out_type=jax.ShapeDtypeStruct((NC, N_ACC, HID), jnp.float32),
        mesh=_sc_mesh(),
        compiler_params=pltpu.CompilerParams(use_tc_tiling_on_sc=False),
        scratch_types=[pltpu.VMEM((NCHUNK, CH), jnp.int32)]
        + [pltpu.VMEM((CH, HID), jnp.float32)] * NBUF
        + [pltpu.SemaphoreType.DMA] * (2 * NBUF)
        + [pltpu.VMEM_SHARED((N_ACC, HID), jnp.float32)],
    )


def _ks(msg, dst3, zeros_acc):
    return _ks_fn()(msg, dst3, zeros_acc)


def _kdeg_body(ones_hbm, dst3_hbm, zeros_hbm, out_hbm, idx_v, rows_v, asem,
               acc_sh):
    cid = lax.axis_index("c")
    sid = lax.axis_index("s")
    wid = sid * NC + cid
    pltpu.sync_copy(zeros_hbm.at[pl.ds(sid * RPT, RPT)],
                    acc_sh.at[pl.ds(sid * RPT, RPT)])
    pltpu.sync_copy(dst3_hbm.at[wid], idx_v)
    pltpu.sync_copy(ones_hbm, rows_v)
    plsc.subcore_barrier()

    def fire(j, carry):
        pltpu.make_async_copy(
            rows_v, acc_sh.at[idx_v.at[j]], asem).start(add=True)
        return carry

    def drain(j, carry):
        pltpu.make_async_copy(rows_v, acc_sh.at[idx_v.at[0]], asem).wait()
        return carry

    lax.fori_loop(0, NCHUNK, fire, 0)
    lax.fori_loop(0, NCHUNK, drain, 0)
    plsc.subcore_barrier()
    pltpu.sync_copy(acc_sh.at[pl.ds(sid * RPT, RPT)],
                    out_hbm.at[cid, pl.ds(sid * RPT, RPT)])


@functools.cache
def _kdeg_fn():
    return pl.kernel(
        _kdeg_body,
        out_type=jax.ShapeDtypeStruct((NC, N_ACC, HID), jnp.float32),
        mesh=_sc_mesh(),
        compiler_params=pltpu.CompilerParams(use_tc_tiling_on_sc=False),
        scratch_types=[
            pltpu.VMEM((NCHUNK, CH), jnp.int32),
            pltpu.VMEM((CH, HID), jnp.float32),
            pltpu.SemaphoreType.DMA,
            pltpu.VMEM_SHARED((N_ACC, HID), jnp.float32),
        ],
    )


def _kdeg(ones_rows, dst3, zeros_acc):
    return _kdeg_fn()(ones_rows, dst3, zeros_acc)


def kernel(x, edge_index, edge_attr, W0, b0, A1, c1, A2, c2, Wroot, bconv,
           Wih, bih, Whh, bhh, W1, b1, W2, b2):
    f32 = jnp.float32
    src = edge_index[0]
    dst = edge_index[1]
    pad = E_PAD - E
    srcp = jnp.pad(src, (0, pad))
    dstp = jnp.pad(dst, (0, pad), constant_values=N)
    eap = jnp.pad(edge_attr, ((0, pad), (0, 0)))
    src3 = srcp.reshape(NW, NCHUNK, CH)
    dst3 = dstp.reshape(NW, NCHUNK, CH)

    w0t = W0.T
    b0row = b0.reshape(1, HID)
    rt = A2.reshape(N_STEPS, HID, HID, MLP_H)
    ball = jnp.concatenate(
        [rt, c2.reshape(N_STEPS, HID, HID)[..., None]], axis=-1)
    bcat = ball.transpose(0, 1, 3, 2).reshape(N_STEPS, HID, M9 * HID)
    a1aug = jnp.concatenate(
        [jnp.swapaxes(A1, 1, 2), jnp.zeros((N_STEPS, E_D, 1), f32)], axis=2)
    c1aug = jnp.concatenate(
        [c1, jnp.ones((N_STEPS, 1), f32)], axis=1)
    a1rept = jnp.repeat(a1aug, HID, axis=2)
    c1rep = jnp.repeat(c1aug, HID, axis=1)[:, None, :]
    wroott = jnp.swapaxes(Wroot, 1, 2)
    bconvrow = bconv.reshape(N_STEPS, 1, HID)
    wiht = Wih.T
    bihrow = bih.reshape(1, 3 * HID)
    whht = Whh.T
    bhhrow = bhh.reshape(1, 3 * HID)
    w1t = W1.T
    b1row = b1.reshape(1, HID // 2)
    w2t = W2.T
    b2row = b2.reshape(1, 1)

    zeros_acc = jnp.zeros((N_ACC, HID), f32)
    ones_rows = jnp.ones((CH, HID), f32)

    xx = _kx0(x, w0t, b0row)
    d = _kdeg(ones_rows, dst3, zeros_acc)
    for l in range(N_STEPS):
        xg = _kg(xx, src3)
        msg = _km(xg, eap, a1rept[l], c1rep[l], bcat[l])
        p = _ks(msg, dst3, zeros_acc)
        xx = _kn(p, d, xx, wroott[l], bconvrow[l], wiht, bihrow, whht,
                 bhhrow)
    return _khead(xx, w1t, b1row, w2t, b2row)


# speedup vs baseline: 2.3963x; 1.2791x over previous
import functools

import jax
import jax.numpy as jnp
from jax import lax
from jax.experimental import pallas as pl
from jax.experimental.pallas import tpu as pltpu
from jax.experimental.pallas import tpu_sc as plsc

N = 10000
E = 160000
IN_NF = 128
HID = 32
E_D = 4
MLP_H = 8
N_STEPS = 3
M9 = MLP_H + 1

NC = 2
NS = 16
NW = NC * NS
CH = 128
E_PAD = 163840
EPW = E_PAD // NW
NCHUNK = EPW // CH
N_ACC = 10240
RPT = N_ACC // NS

@functools.cache
def _sc_mesh():
    return plsc.VectorSubcoreMesh(core_axis_name="c", subcore_axis_name="s",
                                  num_cores=NC, num_subcores=NS)


def _kx0_body(x_ref, w0t_ref, b0_ref, o_ref):
    o_ref[...] = jnp.maximum(
        jnp.dot(x_ref[...], w0t_ref[...], preferred_element_type=jnp.float32, precision=lax.Precision.HIGHEST)
        + b0_ref[...], 0.0)


def _kx0(x, w0t, b0row):
    BN = 2000
    return pl.pallas_call(
        _kx0_body,
        grid=(N // BN,),
        in_specs=[
            pl.BlockSpec((BN, IN_NF), lambda i: (i, 0)),
            pl.BlockSpec((IN_NF, HID), lambda i: (0, 0)),
            pl.BlockSpec((1, HID), lambda i: (0, 0)),
        ],
        out_specs=pl.BlockSpec((BN, HID), lambda i: (i, 0)),
        out_shape=jax.ShapeDtypeStruct((N, HID), jnp.float32),
    )(x, w0t, b0row)


def _km_body(xg_ref, ea_ref, a1rept_ref, c1rep_ref, bcat_ref, o_ref):
    ea = ea_ref[...]
    a1 = a1rept_ref[...]
    e1rep = c1rep_ref[...] + ea[:, 0:1] * a1[0:1, :]
    for k in range(1, E_D):
        e1rep = e1rep + ea[:, k:k + 1] * a1[k:k + 1, :]
    e1rep = jnp.maximum(e1rep, 0.0)
    t = e1rep * jnp.dot(xg_ref[...], bcat_ref[...],
                        preferred_element_type=jnp.float32, precision=lax.Precision.HIGHEST)
    acc = t[:, :HID]
    for m in range(1, M9):
        acc = acc + t[:, m * HID:(m + 1) * HID]
    o_ref[...] = acc


def _km(xg, eap, a1rept, c1rep, bcat):
    BE = 2048
    return pl.pallas_call(
        _km_body,
        grid=(E_PAD // BE,),
        in_specs=[
            pl.BlockSpec((BE, HID), lambda i: (i, 0)),
            pl.BlockSpec((BE, E_D), lambda i: (i, 0)),
            pl.BlockSpec((E_D, M9 * HID), lambda i: (0, 0)),
            pl.BlockSpec((1, M9 * HID), lambda i: (0, 0)),
            pl.BlockSpec((HID, M9 * HID), lambda i: (0, 0)),
        ],
        out_specs=pl.BlockSpec((BE, HID), lambda i: (i, 0)),
        out_shape=jax.ShapeDtypeStruct((E_PAD, HID), jnp.float32),
    )(xg, eap, a1rept, c1rep, bcat)


def _kn_body(p_ref, d_ref, xx_ref, wroott_ref, bconv_ref,
             wiht_ref, bih_ref, whht_ref, bhh_ref, o_ref):
    xx = xx_ref[...]
    deg = jnp.maximum(d_ref[0] + d_ref[1], 1.0)
    agg = (p_ref[0] + p_ref[1]) / deg
    m = jnp.maximum(
        agg + jnp.dot(xx, wroott_ref[...], preferred_element_type=jnp.float32, precision=lax.Precision.HIGHEST)
        + bconv_ref[...], 0.0)
    gi = jnp.dot(m, wiht_ref[...], preferred_element_type=jnp.float32, precision=lax.Precision.HIGHEST) \
        + bih_ref[...]
    gh = jnp.dot(xx, whht_ref[...], preferred_element_type=jnp.float32, precision=lax.Precision.HIGHEST) \
        + bhh_ref[...]
    r = jax.nn.sigmoid(gi[:, :HID] + gh[:, :HID])
    z = jax.nn.sigmoid(gi[:, HID:2 * HID] + gh[:, HID:2 * HID])
    nn = jnp.tanh(gi[:, 2 * HID:] + r * gh[:, 2 * HID:])
    o_ref[...] = (1.0 - z) * nn + z * xx


def _kn(p, d, xx, wroott, bconvrow, wiht, bihrow, whht, bhhrow):
    BN = 2000
    return pl.pallas_call(
        _kn_body,
        grid=(N // BN,),
        in_specs=[
            pl.BlockSpec((2, BN, HID), lambda i: (0, i, 0)),
            pl.BlockSpec((2, BN, HID), lambda i: (0, i, 0)),
            pl.BlockSpec((BN, HID), lambda i: (i, 0)),
            pl.BlockSpec((HID, HID), lambda i: (0, 0)),
            pl.BlockSpec((1, HID), lambda i: (0, 0)),
            pl.BlockSpec((HID, 3 * HID), lambda i: (0, 0)),
            pl.BlockSpec((1, 3 * HID), lambda i: (0, 0)),
            pl.BlockSpec((HID, 3 * HID), lambda i: (0, 0)),
            pl.BlockSpec((1, 3 * HID), lambda i: (0, 0)),
        ],
        out_specs=pl.BlockSpec((BN, HID), lambda i: (i, 0)),
        out_shape=jax.ShapeDtypeStruct((N, HID), jnp.float32),
    )(p, d, xx, wroott, bconvrow, wiht, bihrow, whht, bhhrow)


def _khead_body(xx_ref, w1t_ref, b1_ref, w2t_ref, b2_ref, o_ref):
    g = jnp.mean(xx_ref[...], axis=0, keepdims=True)
    g = jnp.maximum(
        jnp.dot(g, w1t_ref[...], preferred_element_type=jnp.float32, precision=lax.Precision.HIGHEST)
        + b1_ref[...], 0.0)
    o_ref[...] = jnp.dot(g, w2t_ref[...],
                         preferred_element_type=jnp.float32, precision=lax.Precision.HIGHEST) + b2_ref[...]


def _khead(xx, w1t, b1row, w2t, b2row):
    return pl.pallas_call(
        _khead_body,
        out_shape=jax.ShapeDtypeStruct((1, 1), jnp.float32),
    )(xx, w1t, b1row, w2t, b2row)


def _worker_id():
    return lax.axis_index("s") * NC + lax.axis_index("c")


NBUF = 4
NGRP = NCHUNK // NBUF


def _kg_body(xx_hbm, src3_hbm, out_hbm, idx_v,
             r0, r1, r2, r3, g0, g1, g2, g3, s0, s1, s2, s3):
    rows = (r0, r1, r2, r3)
    gs = (g0, g1, g2, g3)
    ss = (s0, s1, s2, s3)
    wid = _worker_id()
    base = wid * EPW
    pltpu.sync_copy(src3_hbm.at[wid], idx_v)

    def g_start(j, b):
        pltpu.make_async_copy(xx_hbm.at[idx_v.at[j]], rows[b], gs[b]).start()

    def g_wait(b):
        pltpu.make_async_copy(xx_hbm.at[idx_v.at[0]], rows[b], gs[b]).wait()

    def s_start(j, b):
        pltpu.make_async_copy(
            rows[b], out_hbm.at[pl.ds(base + j * CH, CH)], ss[b]).start()

    def s_wait(b):
        pltpu.make_async_copy(
            rows[b], out_hbm.at[pl.ds(base, CH)], ss[b]).wait()

    for b in range(NBUF):
        g_start(b, b)

    def group(g, carry):
        for b in range(NBUF):
            g_wait(b)
            s_start(g * NBUF + b, b)

        @pl.when(g < NGRP - 1)
        def _():
            for b in range(NBUF):
                s_wait(b)
                g_start((g + 1) * NBUF + b, b)

        return carry

    lax.fori_loop(0, NGRP, group, 0)
    for b in range(NBUF):
        s_wait(b)


@functools.cache
def _kg_fn():
    return pl.kernel(
        _kg_body,
        out_type=jax.ShapeDtypeStruct((E_PAD, HID), jnp.float32),
        mesh=_sc_mesh(),
        compiler_params=pltpu.CompilerParams(use_tc_tiling_on_sc=False),
        scratch_types=[pltpu.VMEM((NCHUNK, CH), jnp.int32)]
        + [pltpu.VMEM((CH, HID), jnp.float32)] * NBUF
        + [pltpu.SemaphoreType.DMA] * (2 * NBUF),
    )


def _kg(xx, src3):
    return _kg_fn()(xx, src3)


def _ks_body(msg_hbm, dst3_hbm, zeros_hbm, out_hbm, idx_v,
             r0, r1, r2, r3, l0, l1, l2, l3, a0, a1, a2, a3, acc_sh):
    rows = (r0, r1, r2, r3)
    ls = (l0, l1, l2, l3)
    as_ = (a0, a1, a2, a3)
    cid = lax.axis_index("c")
    sid = lax.axis_index("s")
    wid = sid * NC + cid
    base = wid * EPW
    pltpu.sync_copy(zeros_hbm.at[pl.ds(sid * RPT, RPT)],
                    acc_sh.at[pl.ds(sid * RPT, RPT)])
    pltpu.sync_copy(dst3_hbm.at[wid], idx_v)
    plsc.subcore_barrier()

    def l_start(j, b):
        pltpu.make_async_copy(
            msg_hbm.at[pl.ds(base + j * CH, CH)], rows[b], ls[b]).start()

    def l_wait(b):
        pltpu.make_async_copy(
            msg_hbm.at[pl.ds(base, CH)], rows[b], ls[b]).wait()

    def a_start(j, b):
        pltpu.make_async_copy(
            rows[b], acc_sh.at[idx_v.at[j]], as_[b]).start(add=True)

    def a_wait(b):
        pltpu.make_async_copy(
            rows[b], acc_sh.at[idx_v.at[0]], as_[b]).wait()

    for b in range(NBUF):
        l_start(b, b)

    def group(g, carry):
        for b in range(NBUF):
            l_wait(b)
            a_start(g * NBUF + b, b)

        @pl.when(g < NGRP - 1)
        def _():
            for b in range(NBUF):
                a_wait(b)
                l_start((g + 1) * NBUF + b, b)

        return carry

    lax.fori_loop(0, NGRP, group, 0)
    for b in range(NBUF):
        a_wait(b)
    plsc.subcore_barrier()
    pltpu.sync_copy(acc_sh.at[pl.ds(sid * RPT, RPT)],
                    out_hbm.at[cid, pl.ds(sid * RPT, RPT)])


@functools.cache
def _ks_fn():
    return pl.kernel(
        _ks_body,
        out_type=jax.ShapeDtypeStruct((NC, N_ACC, HID), jnp.float32),
        mesh=_sc_mesh(),
        compiler_params=pltpu.CompilerParams(use_tc_tiling_on_sc=False),
        scratch_types=[pltpu.VMEM((NCHUNK, CH), jnp.int32)]
        + [pltpu.VMEM((CH, HID), jnp.float32)] * NBUF
        + [pltpu.SemaphoreType.DMA] * (2 * NBUF)
        + [pltpu.VMEM_SHARED((N_ACC, HID), jnp.float32)],
    )


def _ks(msg, dst3, zeros_acc):
    return _ks_fn()(msg, dst3, zeros_acc)


def _kdeg_body(ones_hbm, dst3_hbm, zeros_hbm, out_hbm, idx_v, rows_v, asem,
               acc_sh):
    cid = lax.axis_index("c")
    sid = lax.axis_index("s")
    wid = sid * NC + cid
    pltpu.sync_copy(zeros_hbm.at[pl.ds(sid * RPT, RPT)],
                    acc_sh.at[pl.ds(sid * RPT, RPT)])
    pltpu.sync_copy(dst3_hbm.at[wid], idx_v)
    pltpu.sync_copy(ones_hbm, rows_v)
    plsc.subcore_barrier()

    def fire(j, carry):
        pltpu.make_async_copy(
            rows_v, acc_sh.at[idx_v.at[j]], asem).start(add=True)
        return carry

    def drain(j, carry):
        pltpu.make_async_copy(rows_v, acc_sh.at[idx_v.at[0]], asem).wait()
        return carry

    lax.fori_loop(0, NCHUNK, fire, 0)
    lax.fori_loop(0, NCHUNK, drain, 0)
    plsc.subcore_barrier()
    pltpu.sync_copy(acc_sh.at[pl.ds(sid * RPT, RPT)],
                    out_hbm.at[cid, pl.ds(sid * RPT, RPT)])


@functools.cache
def _kdeg_fn():
    return pl.kernel(
        _kdeg_body,
        out_type=jax.ShapeDtypeStruct((NC, N_ACC, HID), jnp.float32),
        mesh=_sc_mesh(),
        compiler_params=pltpu.CompilerParams(use_tc_tiling_on_sc=False),
        scratch_types=[
            pltpu.VMEM((NCHUNK, CH), jnp.int32),
            pltpu.VMEM((CH, HID), jnp.float32),
            pltpu.SemaphoreType.DMA,
            pltpu.VMEM_SHARED((N_ACC, HID), jnp.float32),
        ],
    )


def _kdeg(ones_rows, dst3, zeros_acc):
    return _kdeg_fn()(ones_rows, dst3, zeros_acc)


def kernel(x, edge_index, edge_attr, W0, b0, A1, c1, A2, c2, Wroot, bconv,
           Wih, bih, Whh, bhh, W1, b1, W2, b2):
    f32 = jnp.float32
    src = edge_index[0]
    dst = edge_index[1]
    pad = E_PAD - E
    srcp = jnp.pad(src, (0, pad))
    dstp = jnp.pad(dst, (0, pad), constant_values=N)
    eap = jnp.pad(edge_attr, ((0, pad), (0, 0)))
    src3 = srcp.reshape(NW, NCHUNK, CH)
    dst3 = dstp.reshape(NW, NCHUNK, CH)

    w0t = W0.T
    b0row = b0.reshape(1, HID)
    rt = A2.reshape(N_STEPS, HID, HID, MLP_H)
    ball = jnp.concatenate(
        [rt, c2.reshape(N_STEPS, HID, HID)[..., None]], axis=-1)
    bcat = ball.transpose(0, 1, 3, 2).reshape(N_STEPS, HID, M9 * HID)
    a1aug = jnp.concatenate(
        [jnp.swapaxes(A1, 1, 2), jnp.zeros((N_STEPS, E_D, 1), f32)], axis=2)
    c1aug = jnp.concatenate(
        [c1, jnp.ones((N_STEPS, 1), f32)], axis=1)
    a1rept = jnp.repeat(a1aug, HID, axis=2)
    c1rep = jnp.repeat(c1aug, HID, axis=1)[:, None, :]
    wroott = jnp.swapaxes(Wroot, 1, 2)
    bconvrow = bconv.reshape(N_STEPS, 1, HID)
    wiht = Wih.T
    bihrow = bih.reshape(1, 3 * HID)
    whht = Whh.T
    bhhrow = bhh.reshape(1, 3 * HID)
    w1t = W1.T
    b1row = b1.reshape(1, HID // 2)
    w2t = W2.T
    b2row = b2.reshape(1, 1)

    zeros_acc = jnp.zeros((N_ACC, HID), f32)
    ones_rows = jnp.ones((CH, HID), f32)

    xx = _kx0(x, w0t, b0row)
    d = _kdeg(ones_rows, dst3, zeros_acc)
    for l in range(N_STEPS):
        xg = _kg(xx, src3)
        msg = _km(xg, eap, a1rept[l], c1rep[l], bcat[l])
        p = _ks(msg, dst3, zeros_acc)
        xx = _kn(p, d, xx, wroott[l], bconvrow[l], wiht, bihrow, whht,
                 bhhrow)
    return _khead(xx, w1t, b1row, w2t, b2row)


# speedup vs baseline: 2.7226x; 1.1362x over previous
import functools

import jax
import jax.numpy as jnp
from jax import lax
from jax.experimental import pallas as pl
from jax.experimental.pallas import tpu as pltpu
from jax.experimental.pallas import tpu_sc as plsc

N = 10000
E = 160000
IN_NF = 128
HID = 32
E_D = 4
MLP_H = 8
N_STEPS = 3
M9 = MLP_H + 1

NC = 2
NS = 16
NW = NC * NS
CH = 125
EPW = E // NW
NCHUNK = EPW // CH
N_ACC = N
RPT = N_ACC // NS

@functools.cache
def _sc_mesh():
    return plsc.VectorSubcoreMesh(core_axis_name="c", subcore_axis_name="s",
                                  num_cores=NC, num_subcores=NS)


def _kx0_body(x_ref, w0t_ref, b0_ref, o_ref):
    o_ref[...] = jnp.maximum(
        jnp.dot(x_ref[...], w0t_ref[...], preferred_element_type=jnp.float32, precision=lax.Precision.HIGHEST)
        + b0_ref[...], 0.0)


def _kx0(x, w0t, b0row):
    BN = 2000
    return pl.pallas_call(
        _kx0_body,
        grid=(N // BN,),
        in_specs=[
            pl.BlockSpec((BN, IN_NF), lambda i: (i, 0)),
            pl.BlockSpec((IN_NF, HID), lambda i: (0, 0)),
            pl.BlockSpec((1, HID), lambda i: (0, 0)),
        ],
        out_specs=pl.BlockSpec((BN, HID), lambda i: (i, 0)),
        out_shape=jax.ShapeDtypeStruct((N, HID), jnp.float32),
    )(x, w0t, b0row)


def _km_body(xg_ref, ea_ref, a1rept_ref, c1rep_ref, bcat_ref, o_ref):
    ea = ea_ref[...]
    a1 = a1rept_ref[...]
    e1rep = c1rep_ref[...] + ea[:, 0:1] * a1[0:1, :]
    for k in range(1, E_D):
        e1rep = e1rep + ea[:, k:k + 1] * a1[k:k + 1, :]
    e1rep = jnp.maximum(e1rep, 0.0)
    t = e1rep * jnp.dot(xg_ref[...], bcat_ref[...],
                        preferred_element_type=jnp.float32, precision=lax.Precision.HIGHEST)
    acc = t[:, :HID]
    for m in range(1, M9):
        acc = acc + t[:, m * HID:(m + 1) * HID]
    o_ref[...] = acc


def _km(xg, eap, a1rept, c1rep, bcat):
    BE = 2000
    return pl.pallas_call(
        _km_body,
        grid=(E // BE,),
        in_specs=[
            pl.BlockSpec((BE, HID), lambda i: (i, 0)),
            pl.BlockSpec((BE, E_D), lambda i: (i, 0)),
            pl.BlockSpec((E_D, M9 * HID), lambda i: (0, 0)),
            pl.BlockSpec((1, M9 * HID), lambda i: (0, 0)),
            pl.BlockSpec((HID, M9 * HID), lambda i: (0, 0)),
        ],
        out_specs=pl.BlockSpec((BE, HID), lambda i: (i, 0)),
        out_shape=jax.ShapeDtypeStruct((E, HID), jnp.float32),
    )(xg, eap, a1rept, c1rep, bcat)


def _kn_body(p_ref, d_ref, xx_ref, wroott_ref, bconv_ref,
             wiht_ref, bih_ref, whht_ref, bhh_ref, o_ref):
    xx = xx_ref[...]
    deg = jnp.maximum(d_ref[0] + d_ref[1], 1.0)
    agg = (p_ref[0] + p_ref[1]) / deg
    m = jnp.maximum(
        agg + jnp.dot(xx, wroott_ref[...], preferred_element_type=jnp.float32, precision=lax.Precision.HIGHEST)
        + bconv_ref[...], 0.0)
    gi = jnp.dot(m, wiht_ref[...], preferred_element_type=jnp.float32, precision=lax.Precision.HIGHEST) \
        + bih_ref[...]
    gh = jnp.dot(xx, whht_ref[...], preferred_element_type=jnp.float32, precision=lax.Precision.HIGHEST) \
        + bhh_ref[...]
    r = jax.nn.sigmoid(gi[:, :HID] + gh[:, :HID])
    z = jax.nn.sigmoid(gi[:, HID:2 * HID] + gh[:, HID:2 * HID])
    nn = jnp.tanh(gi[:, 2 * HID:] + r * gh[:, 2 * HID:])
    o_ref[...] = (1.0 - z) * nn + z * xx


def _kn(p, d, xx, wroott, bconvrow, wiht, bihrow, whht, bhhrow):
    BN = 2000
    return pl.pallas_call(
        _kn_body,
        grid=(N // BN,),
        in_specs=[
            pl.BlockSpec((2, BN, HID), lambda i: (0, i, 0)),
            pl.BlockSpec((2, BN, HID), lambda i: (0, i, 0)),
            pl.BlockSpec((BN, HID), lambda i: (i, 0)),
            pl.BlockSpec((HID, HID), lambda i: (0, 0)),
            pl.BlockSpec((1, HID), lambda i: (0, 0)),
            pl.BlockSpec((HID, 3 * HID), lambda i: (0, 0)),
            pl.BlockSpec((1, 3 * HID), lambda i: (0, 0)),
            pl.BlockSpec((HID, 3 * HID), lambda i: (0, 0)),
            pl.BlockSpec((1, 3 * HID), lambda i: (0, 0)),
        ],
        out_specs=pl.BlockSpec((BN, HID), lambda i: (i, 0)),
        out_shape=jax.ShapeDtypeStruct((N, HID), jnp.float32),
    )(p, d, xx, wroott, bconvrow, wiht, bihrow, whht, bhhrow)


def _khead_body(xx_ref, w1t_ref, b1_ref, w2t_ref, b2_ref, o_ref):
    g = jnp.mean(xx_ref[...], axis=0, keepdims=True)
    g = jnp.maximum(
        jnp.dot(g, w1t_ref[...], preferred_element_type=jnp.float32, precision=lax.Precision.HIGHEST)
        + b1_ref[...], 0.0)
    o_ref[...] = jnp.dot(g, w2t_ref[...],
                         preferred_element_type=jnp.float32, precision=lax.Precision.HIGHEST) + b2_ref[...]


def _khead(xx, w1t, b1row, w2t, b2row):
    return pl.pallas_call(
        _khead_body,
        out_shape=jax.ShapeDtypeStruct((1, 1), jnp.float32),
    )(xx, w1t, b1row, w2t, b2row)


def _worker_id():
    return lax.axis_index("s") * NC + lax.axis_index("c")


NBUF = 4
NGRP = NCHUNK // NBUF


def _kg_body(xx_hbm, src3_hbm, out_hbm, idx_v,
             r0, r1, r2, r3, g0, g1, g2, g3, s0, s1, s2, s3):
    rows = (r0, r1, r2, r3)
    gs = (g0, g1, g2, g3)
    ss = (s0, s1, s2, s3)
    wid = _worker_id()
    base = wid * EPW
    pltpu.sync_copy(src3_hbm.at[wid], idx_v)

    def g_start(j, b):
        pltpu.make_async_copy(xx_hbm.at[idx_v.at[j]], rows[b], gs[b]).start()

    def g_wait(b):
        pltpu.make_async_copy(xx_hbm.at[idx_v.at[0]], rows[b], gs[b]).wait()

    def s_start(j, b):
        pltpu.make_async_copy(
            rows[b], out_hbm.at[pl.ds(base + j * CH, CH)], ss[b]).start()

    def s_wait(b):
        pltpu.make_async_copy(
            rows[b], out_hbm.at[pl.ds(base, CH)], ss[b]).wait()

    for b in range(NBUF):
        g_start(b, b)

    def group(g, carry):
        for b in range(NBUF):
            g_wait(b)
            s_start(g * NBUF + b, b)

        @pl.when(g < NGRP - 1)
        def _():
            for b in range(NBUF):
                s_wait(b)
                g_start((g + 1) * NBUF + b, b)

        return carry

    lax.fori_loop(0, NGRP, group, 0)
    for b in range(NBUF):
        s_wait(b)


@functools.cache
def _kg_fn():
    return pl.kernel(
        _kg_body,
        out_type=jax.ShapeDtypeStruct((E, HID), jnp.float32),
        mesh=_sc_mesh(),
        compiler_params=pltpu.CompilerParams(use_tc_tiling_on_sc=False),
        scratch_types=[pltpu.VMEM((NCHUNK, CH), jnp.int32)]
        + [pltpu.VMEM((CH, HID), jnp.float32)] * NBUF
        + [pltpu.SemaphoreType.DMA] * (2 * NBUF),
    )


def _kg(xx, src3):
    return _kg_fn()(xx, src3)


def _ks_body(msg_hbm, dst3_hbm, zeros_hbm, out_hbm, idx_v,
             r0, r1, r2, r3, l0, l1, l2, l3, a0, a1, a2, a3, acc_sh):
    rows = (r0, r1, r2, r3)
    ls = (l0, l1, l2, l3)
    as_ = (a0, a1, a2, a3)
    cid = lax.axis_index("c")
    sid = lax.axis_index("s")
    wid = sid * NC + cid
    base = wid * EPW
    pltpu.sync_copy(zeros_hbm.at[pl.ds(sid * RPT, RPT)],
                    acc_sh.at[pl.ds(sid * RPT, RPT)])
    pltpu.sync_copy(dst3_hbm.at[wid], idx_v)
    plsc.subcore_barrier()

    def l_start(j, b):
        pltpu.make_async_copy(
            msg_hbm.at[pl.ds(base + j * CH, CH)], rows[b], ls[b]).start()

    def l_wait(b):
        pltpu.make_async_copy(
            msg_hbm.at[pl.ds(base, CH)], rows[b], ls[b]).wait()

    def a_start(j, b):
        pltpu.make_async_copy(
            rows[b], acc_sh.at[idx_v.at[j]], as_[b]).start(add=True)

    def a_wait(b):
        pltpu.make_async_copy(
            rows[b], acc_sh.at[idx_v.at[0]], as_[b]).wait()

    for b in range(NBUF):
        l_start(b, b)

    def group(g, carry):
        for b in range(NBUF):
            l_wait(b)
            a_start(g * NBUF + b, b)

        @pl.when(g < NGRP - 1)
        def _():
            for b in range(NBUF):
                a_wait(b)
                l_start((g + 1) * NBUF + b, b)

        return carry

    lax.fori_loop(0, NGRP, group, 0)
    for b in range(NBUF):
        a_wait(b)
    plsc.subcore_barrier()
    pltpu.sync_copy(acc_sh.at[pl.ds(sid * RPT, RPT)],
                    out_hbm.at[cid, pl.ds(sid * RPT, RPT)])


@functools.cache
def _ks_fn():
    return pl.kernel(
        _ks_body,
        out_type=jax.ShapeDtypeStruct((NC, N_ACC, HID), jnp.float32),
        mesh=_sc_mesh(),
        compiler_params=pltpu.CompilerParams(use_tc_tiling_on_sc=False),
        scratch_types=[pltpu.VMEM((NCHUNK, CH), jnp.int32)]
        + [pltpu.VMEM((CH, HID), jnp.float32)] * NBUF
        + [pltpu.SemaphoreType.DMA] * (2 * NBUF)
        + [pltpu.VMEM_SHARED((N_ACC, HID), jnp.float32)],
    )


def _ks(msg, dst3, zeros_acc):
    return _ks_fn()(msg, dst3, zeros_acc)


def kernel(x, edge_index, edge_attr, W0, b0, A1, c1, A2, c2, Wroot, bconv,
           Wih, bih, Whh, bhh, W1, b1, W2, b2):
    f32 = jnp.float32
    src3 = edge_index[0].reshape(NW, NCHUNK, CH)
    dst3 = edge_index[1].reshape(NW, NCHUNK, CH)
    eap = edge_attr

    w0t = W0.T
    b0row = b0.reshape(1, HID)
    rt = A2.reshape(N_STEPS, HID, HID, MLP_H)
    ball = jnp.concatenate(
        [rt, c2.reshape(N_STEPS, HID, HID)[..., None]], axis=-1)
    bcat = ball.transpose(0, 1, 3, 2).reshape(N_STEPS, HID, M9 * HID)
    a1aug = jnp.concatenate(
        [jnp.swapaxes(A1, 1, 2), jnp.zeros((N_STEPS, E_D, 1), f32)], axis=2)
    c1aug = jnp.concatenate(
        [c1, jnp.ones((N_STEPS, 1), f32)], axis=1)
    a1rept = jnp.repeat(a1aug, HID, axis=2)
    c1rep = jnp.repeat(c1aug, HID, axis=1)[:, None, :]
    wroott = jnp.swapaxes(Wroot, 1, 2)
    bconvrow = bconv.reshape(N_STEPS, 1, HID)
    wiht = Wih.T
    bihrow = bih.reshape(1, 3 * HID)
    whht = Whh.T
    bhhrow = bhh.reshape(1, 3 * HID)
    w1t = W1.T
    b1row = b1.reshape(1, HID // 2)
    w2t = W2.T
    b2row = b2.reshape(1, 1)

    zeros_acc = jnp.zeros((N_ACC, HID), f32)
    ones_e = jnp.ones((E, HID), f32)

    xx = _kx0(x, w0t, b0row)
    d = _ks(ones_e, dst3, zeros_acc)
    for l in range(N_STEPS):
        xg = _kg(xx, src3)
        msg = _km(xg, eap, a1rept[l], c1rep[l], bcat[l])
        p = _ks(msg, dst3, zeros_acc)
        xx = _kn(p, d, xx, wroott[l], bconvrow[l], wiht, bihrow, whht,
                 bhhrow)
    return _khead(xx, w1t, b1row, w2t, b2row)
